# Initial kernel scaffold; baseline (speedup 1.0000x reference)
import jax
import jax.numpy as jnp
from jax.experimental import pallas as pl


def kernel(x_lidar, x_radar1, x_radar2, edge_index_ll, edge_index_rr1, edge_index_rr2, edge_index_cross, edge_attr_ll, edge_attr_rr1, edge_attr_rr2, edge_attr_cross, params):
    raise NotImplementedError("write your pallas kernel here")


# speedup vs baseline: 30.1127x; 30.1127x over previous
import functools

import jax
import jax.numpy as jnp
from jax import lax
from jax.experimental import pallas as pl
from jax.experimental.pallas import tpu as pltpu
from jax.experimental.pallas import tpu_sc as plsc

F32 = jnp.float32
HEADS, HEAD_DIM, HID, EDGE_DIM = 4, 32, 128, 4
NEG_SLOPE = 0.2
MSG_W = 80
EPAD_Q = 1024
ZR = 112


def _ceil_to(x, q):
    return ((x + q - 1) // q) * q


def _linear_body(x_ref, w_ref, b_ref, o_ref, *, act):
    y = jnp.dot(x_ref[...], w_ref[...], preferred_element_type=F32) + b_ref[...]
    if act == "elu":
        y = jnp.where(y > 0, y, jnp.exp(jnp.minimum(y, 0.0)) - 1.0)
    o_ref[...] = y


NBS = 896


def _tc_linear(x, W, b, act=None):
    n, k = x.shape
    m = W.shape[1]
    bs = NBS
    return pl.pallas_call(
        functools.partial(_linear_body, act=act),
        grid=(n // bs,),
        in_specs=[
            pl.BlockSpec((bs, k), lambda i: (i, 0)),
            pl.BlockSpec((k, m), lambda i: (0, 0)),
            pl.BlockSpec((1, m), lambda i: (0, 0)),
        ],
        out_specs=pl.BlockSpec((bs, m), lambda i: (i, 0)),
        out_shape=jax.ShapeDtypeStruct((n, m), F32),
    )(x, W, b.reshape(1, m))


def _edge_body(xj_ref, xi_ref, ea_ref, we_ref, be_ref, a_ref, b_ref,
               plo_ref, phi_ref, o_ref):
    xj = xj_ref[...]
    ee = jnp.dot(ea_ref[...], we_ref[...], preferred_element_type=F32) + be_ref[...]
    s = xj + xi_ref[...] + ee
    m = jnp.where(s > 0, s, NEG_SLOPE * s)
    alpha = jnp.dot(m, a_ref[...], preferred_element_type=F32)
    ex = jnp.exp(alpha)
    exb = jnp.dot(ex, b_ref[...], preferred_element_type=F32)
    msg = xj * exb
    o_ref[0, :, :64] = msg[:, :64]
    o_ref[0, :, 64:] = jnp.dot(ex, plo_ref[...], preferred_element_type=F32)
    o_ref[1, :, :64] = msg[:, 64:]
    o_ref[1, :, 64:] = jnp.dot(ex, phi_ref[...], preferred_element_type=F32)


def _tc_edge(xj, xi, ea_pad, We, be, att):
    epad = xj.shape[0]
    bs = 1024
    heads_of = jnp.arange(HID) // HEAD_DIM
    A = jnp.zeros((HID, HEADS), F32).at[jnp.arange(HID), heads_of].set(att.reshape(-1))
    B = (heads_of[None, :] == jnp.arange(HEADS)[:, None]).astype(F32)
    Plo = jnp.zeros((HEADS, 16), F32).at[jnp.array([0, 1]), jnp.array([0, 1])].set(1.0)
    Phi = jnp.zeros((HEADS, 16), F32).at[jnp.array([2, 3]), jnp.array([0, 1])].set(1.0)
    return pl.pallas_call(
        _edge_body,
        grid=(epad // bs,),
        in_specs=[
            pl.BlockSpec((bs, HID), lambda i: (i, 0)),
            pl.BlockSpec((bs, HID), lambda i: (i, 0)),
            pl.BlockSpec((bs, EDGE_DIM), lambda i: (i, 0)),
            pl.BlockSpec((EDGE_DIM, HID), lambda i: (0, 0)),
            pl.BlockSpec((1, HID), lambda i: (0, 0)),
            pl.BlockSpec((HID, HEADS), lambda i: (0, 0)),
            pl.BlockSpec((HEADS, HID), lambda i: (0, 0)),
            pl.BlockSpec((HEADS, 16), lambda i: (0, 0)),
            pl.BlockSpec((HEADS, 16), lambda i: (0, 0)),
        ],
        out_specs=pl.BlockSpec((2, bs, MSG_W), lambda i: (0, i, 0)),
        out_shape=jax.ShapeDtypeStruct((2, epad, MSG_W), F32),
    )(xj, xi, ea_pad, We, be.reshape(1, HID), A, B, Plo, Phi)


def _post_body(*refs, nconv, n, bs):
    acc_refs = refs[:nconv]
    bias_refs = refs[nconv:2 * nconv]
    u_ref, ps_ref, pss_ref = refs[2 * nconv:]
    q = None
    for a_ref, b_ref in zip(acc_refs, bias_refs):
        lo = a_ref[0]
        hi = a_ref[1]
        h0 = lo[:, 0:32] / (lo[:, 64:65] + 1e-16)
        h1 = lo[:, 32:64] / (lo[:, 65:66] + 1e-16)
        h2 = hi[:, 0:32] / (hi[:, 64:65] + 1e-16)
        h3 = hi[:, 32:64] / (hi[:, 65:66] + 1e-16)
        qq = jnp.concatenate([h0, h1, h2, h3], axis=1) + b_ref[...]
        q = qq if q is None else q + qq
    u = jnp.where(q > 0, q, jnp.exp(jnp.minimum(q, 0.0)) - 1.0)
    u_ref[...] = u
    row = pl.program_id(0) * bs + lax.broadcasted_iota(jnp.int32, (bs, 1), 0)
    um = jnp.where(row < n, u, 0.0)
    ps_ref[...] = jnp.sum(um, axis=0, keepdims=True).reshape(1, 1, HID)
    pss_ref[...] = jnp.sum(um * um, axis=0, keepdims=True).reshape(1, 1, HID)


def _tc_post1(accs, biases, n, npad):
    bs = NBS
    nblk = npad // bs
    nconv = len(accs)
    in_specs = (
        [pl.BlockSpec((2, bs, MSG_W), lambda i: (0, i, 0)) for _ in accs]
        + [pl.BlockSpec((1, HID), lambda i: (0, 0)) for _ in biases]
    )
    return pl.pallas_call(
        functools.partial(_post_body, nconv=nconv, n=n, bs=bs),
        grid=(nblk,),
        in_specs=in_specs,
        out_specs=[
            pl.BlockSpec((bs, HID), lambda i: (i, 0)),
            pl.BlockSpec((1, 1, HID), lambda i: (i, 0, 0)),
            pl.BlockSpec((1, 1, HID), lambda i: (i, 0, 0)),
        ],
        out_shape=[
            jax.ShapeDtypeStruct((npad, HID), F32),
            jax.ShapeDtypeStruct((nblk, 1, HID), F32),
            jax.ShapeDtypeStruct((nblk, 1, HID), F32),
        ],
    )(*accs, *[b.reshape(1, HID) for b in biases])


def _post2_body(u_ref, ps_ref, pss_ref, g_ref, b_ref, o_ref, mu_ref, sg_ref, *, n):
    @pl.when(pl.program_id(0) == 0)
    def _():
        mu = jnp.sum(ps_ref[:, 0, :], axis=0, keepdims=True) / n
        ms = jnp.sum(pss_ref[:, 0, :], axis=0, keepdims=True) / n
        mu_ref[...] = mu
        sg_ref[...] = lax.rsqrt(ms - mu * mu + 1e-5)

    o_ref[...] = (u_ref[...] - mu_ref[...]) * sg_ref[...] * g_ref[...] + b_ref[...]


def _tc_post2(u, ps, pss, gamma, beta, n):
    npad = u.shape[0]
    bs = NBS
    nblk = ps.shape[0]
    return pl.pallas_call(
        functools.partial(_post2_body, n=n),
        grid=(npad // bs,),
        in_specs=[
            pl.BlockSpec((bs, HID), lambda i: (i, 0)),
            pl.BlockSpec((nblk, 1, HID), lambda i: (0, 0, 0)),
            pl.BlockSpec((nblk, 1, HID), lambda i: (0, 0, 0)),
            pl.BlockSpec((1, HID), lambda i: (0, 0)),
            pl.BlockSpec((1, HID), lambda i: (0, 0)),
        ],
        out_specs=pl.BlockSpec((bs, HID), lambda i: (i, 0)),
        out_shape=jax.ShapeDtypeStruct((npad, HID), F32),
        scratch_shapes=[pltpu.VMEM((1, HID), F32), pltpu.VMEM((1, HID), F32)],
    )(u, ps, pss, gamma.reshape(1, HID), beta.reshape(1, HID))


def _head_body(x_ref, w_ref, b_ref, o_ref):
    o_ref[...] = jnp.sum(x_ref[...] * w_ref[...], axis=1, keepdims=True) + b_ref[...]


def _tc_head(h, W, b):
    npad = h.shape[0]
    bs = NBS
    return pl.pallas_call(
        _head_body,
        grid=(npad // bs,),
        in_specs=[
            pl.BlockSpec((bs, HID), lambda i: (i, 0)),
            pl.BlockSpec((1, HID), lambda i: (0, 0)),
            pl.BlockSpec((1, 1), lambda i: (0, 0)),
        ],
        out_specs=pl.BlockSpec((bs, 1), lambda i: (i, 0)),
        out_shape=jax.ShapeDtypeStruct((npad, 1), F32),
    )(h, W.reshape(1, HID), b.reshape(1, 1))


def _sc_gather(table_l, table_r, src2d, dst2d, epad):
    total_chunks = epad // EPAD_Q
    iters = (total_chunks + 31) // 32
    mesh = plsc.VectorSubcoreMesh(core_axis_name="c", subcore_axis_name="s")

    @functools.partial(
        pl.kernel,
        out_type=(
            jax.ShapeDtypeStruct((epad, HID), F32),
            jax.ShapeDtypeStruct((epad, HID), F32),
        ),
        mesh=mesh,
        scratch_types=[
            pltpu.VMEM((8, 128), jnp.int32),
            pltpu.VMEM((8, 128), jnp.int32),
            pltpu.VMEM((256, HID), F32),
            pltpu.VMEM((256, HID), F32),
            pltpu.SemaphoreType.DMA,
            pltpu.SemaphoreType.DMA,
        ],
        compiler_params=pltpu.CompilerParams(use_tc_tiling_on_sc=False),
    )
    def k(tl_h, tr_h, src_h, dst_h, xj_h, xi_h, sidx, didx, bufj, bufi, semj, semi):
        wid = lax.axis_index("s") * 2 + lax.axis_index("c")

        def body(i, carry):
            chunk = wid + i * 32

            @pl.when(chunk < total_chunks)
            def _():
                row0 = chunk * 8
                pltpu.sync_copy(src_h.at[pl.ds(row0, 8)], sidx)
                pltpu.sync_copy(dst_h.at[pl.ds(row0, 8)], didx)
                for q in range(4):
                    descs = []
                    for j in range(2):
                        descs.append(pltpu.async_copy(
                            tl_h.at[sidx.at[2 * q + j]],
                            bufj.at[pl.ds(j * 128, 128)], semj))
                        descs.append(pltpu.async_copy(
                            tr_h.at[didx.at[2 * q + j]],
                            bufi.at[pl.ds(j * 128, 128)], semi))
                    for d in descs:
                        d.wait()
                    e0 = chunk * EPAD_Q + q * 256
                    pltpu.sync_copy(bufj, xj_h.at[pl.ds(e0, 256)])
                    pltpu.sync_copy(bufi, xi_h.at[pl.ds(e0, 256)])

            return carry

        lax.fori_loop(0, iters, body, 0)

    return k(table_l, table_r, src2d, dst2d)


NDP2 = 12544
SCCH = 512


def _sc_scatter(msg_flat, dst2d, epad, ndp):
    total_chunks = epad // SCCH
    iters = (total_chunks + 15) // 16
    nranges = ndp // NDP2
    rpt = NDP2 // 16
    nz = rpt // ZR
    mesh = plsc.VectorSubcoreMesh(core_axis_name="c", subcore_axis_name="s")

    @functools.partial(
        pl.kernel,
        out_type=jax.ShapeDtypeStruct((2 * ndp, MSG_W), F32),
        mesh=mesh,
        scratch_types=[
            pltpu.VMEM_SHARED((NDP2 + 8, MSG_W), F32),
            pltpu.VMEM((4, 128), jnp.int32),
            pltpu.VMEM((4, 128), jnp.int32),
            pltpu.VMEM((SCCH, MSG_W), F32),
            pltpu.VMEM((ZR, MSG_W), F32),
            pltpu.SemaphoreType.DMA,
        ],
        compiler_params=pltpu.CompilerParams(use_tc_tiling_on_sc=False),
    )
    def k(msg_h, dst_h, out_h, acc, didx, didx2, mbuf, zbuf, sem):
        c = lax.axis_index("c")
        s = lax.axis_index("s")
        zv = jnp.zeros((16,), F32)

        def zb(i, carry):
            for kk in range(MSG_W // 16):
                zbuf[i, pl.ds(kk * 16, 16)] = zv
            return carry

        lax.fori_loop(0, ZR, zb, 0)

        for rng in range(nranges):
            lo = rng * NDP2

            def za(i, carry):
                pltpu.sync_copy(zbuf, acc.at[pl.ds(s * rpt + i * ZR, ZR)])
                return carry

            lax.fori_loop(0, nz, za, 0)
            plsc.subcore_barrier()

            def body(i, carry):
                chunk = s + i * 16

                @pl.when(chunk < total_chunks)
                def _():
                    pltpu.sync_copy(dst_h.at[pl.ds(chunk * 4, 4)], didx)
                    pltpu.sync_copy(
                        msg_h.at[pl.ds(c * epad + chunk * SCCH, SCCH)], mbuf)
                    for j in range(4):
                        for g in range(8):
                            v = didx[j, pl.ds(g * 16, 16)]
                            local = v - lo
                            oob = (local < 0) | (local >= NDP2)
                            didx2[j, pl.ds(g * 16, 16)] = jnp.where(
                                oob, NDP2, local)
                    descs = []
                    for j in range(4):
                        descs.append(pltpu.async_copy(
                            mbuf.at[pl.ds(j * 128, 128)], acc.at[didx2.at[j]],
                            sem, add=True))
                    for d in descs:
                        d.wait()

                return carry

            lax.fori_loop(0, iters, body, 0)
            plsc.subcore_barrier()
            pltpu.sync_copy(
                acc.at[pl.ds(s * rpt, rpt)],
                out_h.at[pl.ds(c * ndp + lo + s * rpt, rpt)])
            if rng + 1 < nranges:
                plsc.subcore_barrier()

    return k(msg_flat, dst2d)


def _pad_edges(ei, ea, num_dst):
    e = ei.shape[1]
    epad = _ceil_to(e, EPAD_Q)
    src = jnp.concatenate(
        [ei[0].astype(jnp.int32), jnp.zeros((epad - e,), jnp.int32)])
    dst = jnp.concatenate(
        [ei[1].astype(jnp.int32),
         jnp.full((epad - e,), num_dst, jnp.int32)])
    ea_pad = jnp.pad(ea, ((0, epad - e), (0, 0)))
    return src.reshape(epad // 128, 128), dst.reshape(epad // 128, 128), ea_pad, epad


def _conv(h_src, h_dst, src2d, dst2d, ea_pad, epad, gp, num_dst, ndp):
    xl = _tc_linear(h_src, gp["lin_l"]["W"], gp["lin_l"]["b"])
    xr = _tc_linear(h_dst, gp["lin_r"]["W"], gp["lin_r"]["b"])
    xj, xi = _sc_gather(xl, xr, src2d, dst2d, epad)
    msg = _tc_edge(xj, xi, ea_pad, gp["lin_e"]["W"], gp["lin_e"]["b"], gp["att"])
    acc = _sc_scatter(msg.reshape(2 * epad, MSG_W), dst2d, epad, ndp)
    return acc.reshape(2, ndp, MSG_W)


def kernel(x_lidar, x_radar1, x_radar2, edge_index_ll, edge_index_rr1,
           edge_index_rr2, edge_index_cross, edge_attr_ll, edge_attr_rr1,
           edge_attr_rr2, edge_attr_cross, params):
    n_l = x_lidar.shape[0]
    n_r1 = x_radar1.shape[0]
    n_r2 = x_radar2.shape[0]
    ndp_l = _ceil_to(n_l + 1, NBS)
    ndp_r = _ceil_to(n_r1 + 1, NBS)

    ell = _pad_edges(edge_index_ll, edge_attr_ll, n_l)
    err1 = _pad_edges(edge_index_rr1, edge_attr_rr1, n_r1)
    err2 = _pad_edges(edge_index_rr2, edge_attr_rr2, n_r2)
    ex_ = _pad_edges(edge_index_cross, edge_attr_cross, n_r1)

    xl_p = jnp.pad(x_lidar, ((0, ndp_l - n_l), (0, 0)))
    xr1_p = jnp.pad(x_radar1, ((0, ndp_r - n_r1), (0, 0)))
    xr2_p = jnp.pad(x_radar2, ((0, ndp_r - n_r2), (0, 0)))

    pr = params["proj"]
    h_l = _tc_linear(xl_p, pr["lidar"]["W"], pr["lidar"]["b"], act="elu")
    h_r1 = _tc_linear(xr1_p, pr["radar1"]["W"], pr["radar1"]["b"], act="elu")
    h_r2 = _tc_linear(xr2_p, pr["radar2"]["W"], pr["radar2"]["b"], act="elu")

    for i in range(len(params["convs"])):
        cp = params["convs"][i]
        bn = params["bn"][i]
        acc_ll = _conv(h_l, h_l, ell[0], ell[1], ell[2], ell[3],
                       cp["ll"], n_l, ndp_l)
        acc_r1 = _conv(h_r1, h_r1, err1[0], err1[1], err1[2], err1[3],
                       cp["rr1"], n_r1, ndp_r)
        acc_x = _conv(h_l, h_r1, ex_[0], ex_[1], ex_[2], ex_[3],
                      cp["cross"], n_r1, ndp_r)
        acc_r2 = _conv(h_r2, h_r2, err2[0], err2[1], err2[2], err2[3],
                       cp["rr2"], n_r2, ndp_r)

        u_l, ps_l, pss_l = _tc_post1([acc_ll], [cp["ll"]["bias"]], n_l, ndp_l)
        h_l = _tc_post2(u_l, ps_l, pss_l, bn["lidar"]["gamma"],
                        bn["lidar"]["beta"], n_l)
        u_r1, ps_r1, pss_r1 = _tc_post1(
            [acc_r1, acc_x], [cp["rr1"]["bias"], cp["cross"]["bias"]], n_r1, ndp_r)
        h_r1 = _tc_post2(u_r1, ps_r1, pss_r1,
                         bn["radar1"]["gamma"], bn["radar1"]["beta"], n_r1)
        u_r2, ps_r2, pss_r2 = _tc_post1([acc_r2], [cp["rr2"]["bias"]], n_r2, ndp_r)
        h_r2 = _tc_post2(u_r2, ps_r2, pss_r2,
                         bn["radar2"]["gamma"], bn["radar2"]["beta"], n_r2)

    out_l = _tc_head(h_l, params["head_lidar"]["W"], params["head_lidar"]["b"])
    out_r1 = _tc_head(h_r1, params["head_radar"]["W"], params["head_radar"]["b"])
    out_r2 = _tc_head(h_r2, params["head_radar"]["W"], params["head_radar"]["b"])
    return (out_l[:n_l], out_r1[:n_r1], out_r2[:n_r2])
